# Initial kernel scaffold; baseline (speedup 1.0000x reference)
import jax
import jax.numpy as jnp
from jax.experimental import pallas as pl


def kernel(time_vector, constant_offset, linear_trend, seasonal_amplitudes, seasonal_phases, neighbor_weights, periods, neighbor_indices):
    raise NotImplementedError("write your pallas kernel here")


# speedup vs baseline: 99.7782x; 99.7782x over previous
import functools
import math

import jax
import jax.numpy as jnp
from jax import lax
from jax.experimental import pallas as pl
from jax.experimental.pallas import tpu as pltpu
from jax.experimental.pallas import tpu_sc as plsc

_N = 10000
_K = 32
_T = 256
_C = 4
_NW = 32
_BPW = 320
_GRP = 16
_NGRP = _BPW // _GRP
_TAIL = _N - (_NW - 1) * _BPW
_BLK = 512
_TWO_PI = 2.0 * math.pi


def _prep_body(ph_ref, t_ref, per_ref, cp_ref, sp_ref, b_ref):
    ph = ph_ref[...]
    cp_ref[...] = jnp.cos(ph)
    sp_ref[...] = jnp.sin(ph)
    t = t_ref[...]
    srows, crows = [], []
    for i in range(_C):
        ang = (_TWO_PI / per_ref[i]) * t
        srows.append(jnp.sin(ang))
        crows.append(jnp.cos(ang))
    b_ref[...] = jnp.concatenate(
        [jnp.ones((1, _T), jnp.float32), t] + srows + crows
        + [jnp.zeros((6, _T), jnp.float32)], axis=0)


def _prep(ph_t, tv, periods):
    return pl.pallas_call(
        _prep_body,
        out_shape=(
            jax.ShapeDtypeStruct((_C, _N), jnp.float32),
            jax.ShapeDtypeStruct((_C, _N), jnp.float32),
            jax.ShapeDtypeStruct((16, _T), jnp.float32),
        ),
        in_specs=[
            pl.BlockSpec(memory_space=pltpu.VMEM),
            pl.BlockSpec(memory_space=pltpu.VMEM),
            pl.BlockSpec(memory_space=pltpu.SMEM),
        ],
        out_specs=(
            pl.BlockSpec(memory_space=pltpu.VMEM),
            pl.BlockSpec(memory_space=pltpu.VMEM),
            pl.BlockSpec(memory_space=pltpu.VMEM),
        ),
    )(ph_t, tv, periods)


_SC_MESH = plsc.VectorSubcoreMesh(core_axis_name="c", subcore_axis_name="s")


@functools.partial(
    pl.kernel,
    mesh=_SC_MESH,
    out_type=jax.ShapeDtypeStruct((_N * 16,), jnp.float32),
    scratch_types=[
        pltpu.VMEM((_BPW * _K,), jnp.int32),
        pltpu.VMEM((_BPW * _K,), jnp.float32),
        pltpu.VMEM((_N,), jnp.float32),
        pltpu.VMEM((_N,), jnp.float32),
        pltpu.VMEM((_N,), jnp.float32),
        pltpu.VMEM((_BPW * 16,), jnp.float32),
    ],
    compiler_params=pltpu.CompilerParams(needs_layout_passes=False),
)
def _sc_smooth(a0, a1, a2, a3, c0, c1, c2, c3, s0, s1, s2, s3,
               idx_hbm, w_hbm, out_hbm,
               idx_v, w_v, amp_v, cp_v, sp_v, out_v):
    amp_tabs = (a0, a1, a2, a3)
    cp_tabs = (c0, c1, c2, c3)
    sp_tabs = (s0, s1, s2, s3)
    wid = lax.axis_index("s") * 2 + lax.axis_index("c")
    is_last = wid == _NW - 1
    base = jnp.minimum(wid * _BPW, _N - _BPW)

    pltpu.sync_copy(idx_hbm.at[pl.ds(base * _K, _BPW * _K)], idx_v)
    pltpu.sync_copy(w_hbm.at[pl.ds(base * _K, _BPW * _K)], w_v)

    lane = lax.iota(jnp.int32, _GRP)

    for c in range(_C):
        pltpu.sync_copy(amp_tabs[c], amp_v)
        pltpu.sync_copy(cp_tabs[c], cp_v)
        pltpu.sync_copy(sp_tabs[c], sp_v)

        def group_body(g, _, c=c):
            gb = g * _GRP
            s_vec = gb + lane
            jj0 = s_vec * _K
            oo = s_vec * 16
            zero = jnp.zeros((_GRP,), jnp.float32)

            def k_step(k, carry):
                wsum, ssum, ssq, wr, wi, jj = carry
                ii = plsc.load_gather(idx_v, [jj])
                ww = plsc.load_gather(w_v, [jj])
                av = plsc.load_gather(amp_v, [ii])
                cv = plsc.load_gather(cp_v, [ii])
                sv = plsc.load_gather(sp_v, [ii])
                return (wsum + av * ww, ssum + av, ssq + av * av,
                        wr + cv * ww, wi + sv * ww, jj + 1)

            wsum, ssum, ssq, wr, wi, _jj = lax.fori_loop(
                0, _K, k_step, (zero, zero, zero, zero, zero, jj0))

            own = amp_v[pl.ds(base + gb, _GRP)]
            mean = ssum * (1.0 / _K)
            var = (ssq - mean * ssum) * (1.0 / (_K - 1))
            alpha = 0.25 / (1.0 + 0.1 * var)
            amp_o = (1.0 - alpha) * own + alpha * wsum
            plsc.store_scatter(out_v, [oo + c], amp_o)
            plsc.store_scatter(out_v, [oo + (4 + c)], wr)
            plsc.store_scatter(out_v, [oo + (8 + c)], wi)
            return 0

        lax.fori_loop(0, _NGRP, group_body, 0)

    @pl.when(jnp.logical_not(is_last))
    def _():
        pltpu.sync_copy(out_v, out_hbm.at[pl.ds(wid * (_BPW * 16), _BPW * 16)])

    @pl.when(is_last)
    def _():
        pltpu.sync_copy(out_v.at[pl.ds((_BPW - _TAIL) * 16, _TAIL * 16)],
                        out_hbm.at[pl.ds((_N - _TAIL) * 16, _TAIL * 16)])


def _synth_body(cc_ref, st_ref, cp_ref, sp_ref, b_ref, out_ref):
    st = st_ref[...].T
    amp = st[0:4, :]
    wr = st[4:8, :]
    wi = st[8:12, :]
    a = 0.15 * jnp.sqrt(wr * wr + wi * wi)
    mr = (1.0 - a) * cp_ref[...] + a * wr
    mi = (1.0 - a) * sp_ref[...] + a * wi
    h2 = mr * mr + mi * mi
    rh = lax.rsqrt(jnp.maximum(h2, 1e-30))
    fa = amp * mr * rh
    fb = amp * mi * rh
    f = jnp.concatenate(
        [cc_ref[...], fa, fb, jnp.zeros((6, _BLK), jnp.float32)], axis=0)
    out_ref[...] = lax.dot_general(
        f, b_ref[...], (((0,), (0,)), ((), ())),
        preferred_element_type=jnp.float32,
        precision=lax.Precision.HIGHEST)


def _synth(cc, stats, cp, sp, basis):
    grid = pl.cdiv(_N, _BLK)
    return pl.pallas_call(
        _synth_body,
        grid=(grid,),
        in_specs=[
            pl.BlockSpec((2, _BLK), lambda i: (0, i)),
            pl.BlockSpec((_BLK, 16), lambda i: (i, 0)),
            pl.BlockSpec((_C, _BLK), lambda i: (0, i)),
            pl.BlockSpec((_C, _BLK), lambda i: (0, i)),
            pl.BlockSpec((16, _T), lambda i: (0, 0)),
        ],
        out_specs=pl.BlockSpec((_BLK, _T), lambda i: (i, 0)),
        out_shape=jax.ShapeDtypeStruct((_N, _T), jnp.float32),
    )(cc, stats, cp, sp, basis)


def kernel(time_vector, constant_offset, linear_trend, seasonal_amplitudes,
           seasonal_phases, neighbor_weights, periods, neighbor_indices):
    ph_t = seasonal_phases.T.astype(jnp.float32)
    idx2 = neighbor_indices.astype(jnp.int32)
    w2 = neighbor_weights.astype(jnp.float32)
    cc = jnp.stack([constant_offset, linear_trend]).astype(jnp.float32)
    tv = time_vector.astype(jnp.float32).reshape(1, _T)

    cp, sp, basis = _prep(ph_t, tv, periods.astype(jnp.float32))
    amp32 = seasonal_amplitudes.astype(jnp.float32)
    amps = [amp32[:, c] for c in range(_C)]
    cps = [cp[c] for c in range(_C)]
    sps = [sp[c] for c in range(_C)]
    stats_flat = _sc_smooth(*amps, *cps, *sps,
                            idx2.reshape(_N * _K), w2.reshape(_N * _K))
    stats = stats_flat.reshape(_N, 16)
    return _synth(cc, stats, cp, sp, basis)
